# baseline (speedup 1.0000x reference)
import jax
import jax.numpy as jnp
import numpy as np
from jax.experimental import pallas as pl

N = 32
BB = 8
M = BB * N
GROUPS = 32


def _dot(a, b, dims):
    return jax.lax.dot_general(a, b, (dims, ((), ())),
                               preferred_element_type=jnp.float32)


def _gat_kernel(e_ref, w_ref, al_ref, ar_ref, b_ref, a_ref, sel_ref, o_ref):
    w = w_ref[...]
    al = al_ref[...]
    ar = ar_ref[...]
    bias = b_ref[...]
    amask = a_ref[...]
    sel = sel_ref[...]

    e2 = [e_ref[i * M:(i + 1) * M, :] for i in range(GROUPS)]
    h = [_dot(x, w, ((1,), (1,))) for x in e2]
    s = [_dot(sel, x, ((1,), (0,))) for x in e2]
    mb = [jnp.sum(x * x, axis=1, keepdims=True) * (1.0 / (N * N)) for x in s]
    thr = [_dot(x, sel, ((0,), (0,))) for x in mb]
    el = [_dot(x, al, ((1,), (1,))) for x in h]
    er = [_dot(ar, x, ((1,), (1,))) for x in h]
    def _edge_probs(gx, tx, ex, rx):
        cond = (gx + amask) > tx
        e_ = ex + rx
        e_ = jnp.maximum(e_, 0.2 * e_)
        return jnp.exp(jnp.where(cond, e_, -1e9))
    hb = [x.astype(jnp.bfloat16) for x in h]
    ones = jnp.ones((M, 1), dtype=jnp.bfloat16)
    for i in range(GROUPS):
        g = _dot(e2[i], e2[i], ((1,), (1,)))
        p = _edge_probs(g, thr[i], el[i], er[i]).astype(jnp.bfloat16)
        den = _dot(p, ones, ((0,), (0,)))
        rst = _dot(p, hb[i], ((0,), (0,)))
        rst = rst * (1.0 / den) + bias
        rst = jnp.where(rst > 0, rst, jnp.exp(rst) - 1.0)
        out = _dot(sel, rst, ((1,), (0,))) * (1.0 / N)
        o_ref[i * BB:(i + 1) * BB, :] = out


def kernel(embedding, W, attn_l, attn_r, bias):
    b, n, d_in = embedding.shape
    d_out = W.shape[0]
    e2 = embedding.reshape(b * n, d_in)
    al = attn_l.reshape(1, d_out)
    ar = attn_r.reshape(1, d_out)
    b2 = bias.reshape(1, d_out)

    rows = np.arange(M)
    same_block = (rows[:, None] // N) == (rows[None, :] // N)
    amask_np = np.where(same_block, 0.0, -1e30).astype(np.float32)
    np.fill_diagonal(amask_np, 1e30)
    amask = jnp.asarray(amask_np)
    sel = jnp.asarray(
        (np.arange(BB)[:, None] == (rows[None, :] // N)).astype(np.float32)
    )

    step = GROUPS * BB
    return pl.pallas_call(
        _gat_kernel,
        grid=(b // step,),
        in_specs=[
            pl.BlockSpec((GROUPS * M, d_in), lambda i: (i, 0)),
            pl.BlockSpec((d_out, d_in), lambda i: (0, 0)),
            pl.BlockSpec((1, d_out), lambda i: (0, 0)),
            pl.BlockSpec((1, d_out), lambda i: (0, 0)),
            pl.BlockSpec((1, d_out), lambda i: (0, 0)),
            pl.BlockSpec((M, M), lambda i: (0, 0)),
            pl.BlockSpec((BB, M), lambda i: (0, 0)),
        ],
        out_specs=pl.BlockSpec((step, d_out), lambda i: (i, 0)),
        out_shape=jax.ShapeDtypeStruct((b, d_out), jnp.float32),
    )(e2, W, al, ar, b2, amask, sel)


# speedup vs baseline: 2.3448x; 2.3448x over previous
import jax
import jax.numpy as jnp
import numpy as np
from jax.experimental import pallas as pl

N = 32
BB = 8
M = BB * N
GROUPS = 32


def _dot(a, b, dims):
    return jax.lax.dot_general(a, b, (dims, ((), ())),
                               preferred_element_type=jnp.float32)


def _gat_kernel(e_ref, w_ref, al_ref, ar_ref, b_ref, a_ref, sel_ref, o_ref):
    w = w_ref[...]
    al = al_ref[...]
    ar = ar_ref[...]
    bias = b_ref[...]
    amask = a_ref[...]
    sel = sel_ref[...]

    e2 = [e_ref[i * M:(i + 1) * M, :] for i in range(GROUPS)]
    h = [_dot(x, w, ((1,), (1,))) for x in e2]
    s = [_dot(sel, x, ((1,), (0,))) for x in e2]
    mb = [jnp.sum(x * x, axis=1, keepdims=True) * (1.0 / (N * N)) for x in s]
    thr = [_dot(x, sel, ((0,), (0,))) for x in mb]
    el = [_dot(x, al, ((1,), (1,))) for x in h]
    er = [_dot(ar, x, ((1,), (1,))) for x in h]
    def _edge_probs(gx, tx, ex, rx):
        cond = (gx + amask) > tx
        e_ = ex + rx
        e_ = jnp.maximum(e_, 0.2 * e_)
        return jnp.exp(jnp.where(cond, e_, -1e9))
    g = [_dot(x, x, ((1,), (1,))) for x in e2]
    p = [_edge_probs(gx, tx, ex, rx).astype(jnp.bfloat16)
         for gx, tx, ex, rx in zip(g, thr, el, er)]
    hb = [x.astype(jnp.bfloat16) for x in h]
    ones = jnp.ones((M, 1), dtype=jnp.bfloat16)
    den = [_dot(x, ones, ((0,), (0,))) for x in p]
    rst = [_dot(x, h_, ((0,), (0,))) for x, h_ in zip(p, hb)]
    rst = [x * (1.0 / d) + bias for x, d in zip(rst, den)]
    rst = [jnp.where(x > 0, x, jnp.exp(x) - 1.0) for x in rst]
    out = [_dot(sel, x, ((1,), (0,))) * (1.0 / N) for x in rst]
    for i in range(GROUPS):
        o_ref[i * BB:(i + 1) * BB, :] = out[i]


def kernel(embedding, W, attn_l, attn_r, bias):
    b, n, d_in = embedding.shape
    d_out = W.shape[0]
    e2 = embedding.reshape(b * n, d_in)
    al = attn_l.reshape(1, d_out)
    ar = attn_r.reshape(1, d_out)
    b2 = bias.reshape(1, d_out)

    rows = np.arange(M)
    same_block = (rows[:, None] // N) == (rows[None, :] // N)
    amask_np = np.where(same_block, 0.0, -1e30).astype(np.float32)
    np.fill_diagonal(amask_np, 1e30)
    amask = jnp.asarray(amask_np)
    sel = jnp.asarray(
        (np.arange(BB)[:, None] == (rows[None, :] // N)).astype(np.float32)
    )

    step = GROUPS * BB
    return pl.pallas_call(
        _gat_kernel,
        grid=(b // step,),
        in_specs=[
            pl.BlockSpec((GROUPS * M, d_in), lambda i: (i, 0)),
            pl.BlockSpec((d_out, d_in), lambda i: (0, 0)),
            pl.BlockSpec((1, d_out), lambda i: (0, 0)),
            pl.BlockSpec((1, d_out), lambda i: (0, 0)),
            pl.BlockSpec((1, d_out), lambda i: (0, 0)),
            pl.BlockSpec((M, M), lambda i: (0, 0)),
            pl.BlockSpec((BB, M), lambda i: (0, 0)),
        ],
        out_specs=pl.BlockSpec((step, d_out), lambda i: (i, 0)),
        out_shape=jax.ShapeDtypeStruct((b, d_out), jnp.float32),
    )(e2, W, al, ar, b2, amask, sel)
